# baseline (speedup 1.0000x reference)
import functools

import jax
import jax.numpy as jnp
from jax.experimental import pallas as pl
from jax.experimental.pallas import tpu as pltpu


@functools.lru_cache(maxsize=None)
def _build_call(batch, h, w, dim):
    def body(col_ref, row_ref, out_ref, img, sem):
        col = col_ref[...]
        row = row_ref[...]
        left = jnp.broadcast_to(col[None, :, :], (h, w, dim))
        nchunk = 16
        ch = h // nchunk
        copies = []
        for c in range(nchunk):
            lo, hi = c * ch, (c + 1) * ch
            right = jnp.broadcast_to(row[lo:hi, None, :], (ch, w, dim))
            img[lo:hi] = jnp.concatenate([left[:ch], right], axis=-1)
            sl = pl.ds(lo, ch)
            for b in range(batch):
                cp = pltpu.make_async_copy(img.at[sl], out_ref.at[b, sl], sem)
                cp.start()
                copies.append(cp)
        for cp in copies:
            cp.wait()

    return pl.pallas_call(
        body,
        out_shape=jax.ShapeDtypeStruct((batch, h, w, 2 * dim), jnp.float32),
        in_specs=[
            pl.BlockSpec(memory_space=pltpu.VMEM),
            pl.BlockSpec(memory_space=pltpu.VMEM),
        ],
        out_specs=pl.BlockSpec(memory_space=pl.ANY),
        scratch_shapes=[
            pltpu.VMEM((h, w, 2 * dim), jnp.float32),
            pltpu.SemaphoreType.DMA,
        ],
    )


def kernel(inputs, row_embed, col_embed):
    batch, h, w, _ = inputs.shape
    dim = col_embed.shape[1]
    return _build_call(batch, h, w, dim)(col_embed[:w], row_embed[:h])


# speedup vs baseline: 1.0099x; 1.0099x over previous
import functools

import jax
import jax.numpy as jnp
from jax.experimental import pallas as pl
from jax.experimental.pallas import tpu as pltpu


@functools.lru_cache(maxsize=None)
def _build_call(batch, h, w, dim):
    def body(col_ref, row_ref, out_ref, img, sem):
        col = col_ref[...]
        row = row_ref[...]
        left = jnp.broadcast_to(col[None, :, :], (h, w, dim))
        nchunk = 4
        ch = h // nchunk
        copies = []
        for c in range(nchunk):
            lo, hi = c * ch, (c + 1) * ch
            right = jnp.broadcast_to(row[lo:hi, None, :], (ch, w, dim))
            img[lo:hi] = jnp.concatenate([left[:ch], right], axis=-1)
            sl = pl.ds(lo, ch)
            for b in range(batch):
                cp = pltpu.make_async_copy(img.at[sl], out_ref.at[b, sl], sem)
                cp.start()
                copies.append(cp)
        for cp in copies:
            cp.wait()

    return pl.pallas_call(
        body,
        out_shape=jax.ShapeDtypeStruct((batch, h, w, 2 * dim), jnp.float32),
        in_specs=[
            pl.BlockSpec(memory_space=pltpu.VMEM),
            pl.BlockSpec(memory_space=pltpu.VMEM),
        ],
        out_specs=pl.BlockSpec(memory_space=pl.ANY),
        scratch_shapes=[
            pltpu.VMEM((h, w, 2 * dim), jnp.float32),
            pltpu.SemaphoreType.DMA,
        ],
    )


def kernel(inputs, row_embed, col_embed):
    batch, h, w, _ = inputs.shape
    dim = col_embed.shape[1]
    return _build_call(batch, h, w, dim)(col_embed[:w], row_embed[:h])


# speedup vs baseline: 1.0148x; 1.0049x over previous
import functools

import jax
import jax.numpy as jnp
from jax.experimental import pallas as pl
from jax.experimental.pallas import tpu as pltpu


@functools.lru_cache(maxsize=None)
def _build_call(batch, h, w, dim):
    def body(col_ref, row_ref, out_ref, img, sem):
        col = col_ref[...]
        row = row_ref[...]
        left = jnp.broadcast_to(col[None, :, :], (h, w, dim))
        nchunk = 8
        ch = h // nchunk
        copies = []
        for c in range(nchunk):
            lo, hi = c * ch, (c + 1) * ch
            right = jnp.broadcast_to(row[lo:hi, None, :], (ch, w, dim))
            img[lo:hi] = jnp.concatenate([left[:ch], right], axis=-1)
            sl = pl.ds(lo, ch)
            for b in range(batch):
                cp = pltpu.make_async_copy(img.at[sl], out_ref.at[b, sl], sem)
                cp.start()
                copies.append(cp)
        for cp in copies:
            cp.wait()

    return pl.pallas_call(
        body,
        out_shape=jax.ShapeDtypeStruct((batch, h, w, 2 * dim), jnp.float32),
        in_specs=[
            pl.BlockSpec(memory_space=pltpu.VMEM),
            pl.BlockSpec(memory_space=pltpu.VMEM),
        ],
        out_specs=pl.BlockSpec(memory_space=pl.ANY),
        scratch_shapes=[
            pltpu.VMEM((h, w, 2 * dim), jnp.float32),
            pltpu.SemaphoreType.DMA,
        ],
    )


def kernel(inputs, row_embed, col_embed):
    batch, h, w, _ = inputs.shape
    dim = col_embed.shape[1]
    return _build_call(batch, h, w, dim)(col_embed[:w], row_embed[:h])


# speedup vs baseline: 1.0169x; 1.0020x over previous
import functools

import jax
import jax.numpy as jnp
from jax.experimental import pallas as pl
from jax.experimental.pallas import tpu as pltpu


@functools.lru_cache(maxsize=None)
def _build_call(batch, h, w, dim):
    def body(col_ref, row_ref, out_ref, img, sem, sem2):
        col = col_ref[...]
        row = row_ref[...]
        left = jnp.broadcast_to(col[None, :, :], (h, w, dim))
        nchunk = 8
        ch = h // nchunk
        copies = []
        for c in range(nchunk):
            lo, hi = c * ch, (c + 1) * ch
            right = jnp.broadcast_to(row[lo:hi, None, :], (ch, w, dim))
            img[lo:hi] = jnp.concatenate([left[:ch], right], axis=-1)
            sl = pl.ds(lo, ch)
            for b in range(batch):
                s = sem if b % 2 == 0 else sem2
                cp = pltpu.make_async_copy(img.at[sl], out_ref.at[b, sl], s)
                cp.start()
                copies.append(cp)
        for cp in copies:
            cp.wait()

    return pl.pallas_call(
        body,
        out_shape=jax.ShapeDtypeStruct((batch, h, w, 2 * dim), jnp.float32),
        in_specs=[
            pl.BlockSpec(memory_space=pltpu.VMEM),
            pl.BlockSpec(memory_space=pltpu.VMEM),
        ],
        out_specs=pl.BlockSpec(memory_space=pl.ANY),
        scratch_shapes=[
            pltpu.VMEM((h, w, 2 * dim), jnp.float32),
            pltpu.SemaphoreType.DMA,
            pltpu.SemaphoreType.DMA,
        ],
    )


def kernel(inputs, row_embed, col_embed):
    batch, h, w, _ = inputs.shape
    dim = col_embed.shape[1]
    return _build_call(batch, h, w, dim)(col_embed[:w], row_embed[:h])
